# baseline (speedup 1.0000x reference)
import functools

import jax
import jax.numpy as jnp
from jax import lax
from jax.experimental import pallas as pl
from jax.experimental.pallas import tpu as pltpu
from jax.experimental.pallas import tpu_sc as plsc

_BATCH = 16384
_NSUB = 4
_NGROUPS = 8
_NT = 16
_PER = _BATCH // _NT
_CHUNKS = _PER // 16
_UNROLL = 4

_mesh = plsc.VectorSubcoreMesh(
    core_axis_name="c", subcore_axis_name="s", num_cores=1)


@functools.partial(
    pl.kernel,
    out_type=jax.ShapeDtypeStruct((16,), jnp.float32),
    mesh=_mesh,
    scratch_types=[
        pltpu.VMEM((2 * _PER,), jnp.float32),
        pltpu.VMEM((_PER,), jnp.int32),
        pltpu.VMEM((_PER,), jnp.int32),
        pltpu.VMEM((256,), jnp.float32),
        pltpu.VMEM((_NT * 256,), jnp.float32),
        pltpu.VMEM((16,), jnp.float32),
        pltpu.VMEM_SHARED((_NT * 256,), jnp.float32),
    ],
    compiler_params=pltpu.CompilerParams(
        needs_layout_passes=False, use_tc_tiling_on_sc=False),
)
def _sc_loss(logits_hbm, tgt_hbm, sg_hbm, out_hbm,
             lg_v, tv, sv, blk, gath, outv, shared):
    wid = lax.axis_index("s")
    base = wid * _PER

    pltpu.sync_copy(logits_hbm.at[pl.ds(2 * base, 2 * _PER)], lg_v)
    pltpu.sync_copy(tgt_hbm.at[pl.ds(base, _PER)], tv)
    pltpu.sync_copy(sg_hbm.at[pl.ds(base, _PER)], sv)

    zeros = jnp.zeros((16,), jnp.float32)
    ones = jnp.ones((16,), jnp.float32)

    def one_chunk(i, accs, accc):
        t = tv[pl.ds(i * 16, 16)]
        s = sv[pl.ds(i * 16, 16)]
        off = (i // 8) * 256 + (i % 8) * 16
        la = lg_v[pl.ds(off, 16)]
        lb = lg_v[pl.ds(off + 128, 16)]
        d = (1 - 2 * t).astype(jnp.float32) * (la - lb)
        m = jnp.maximum(-d, 0.0)
        u = jnp.exp(-jnp.abs(d))
        z = u / (u + 2.0)
        z2 = z * z
        p = 2.0 * z * (1.0 + z2 * (0.33333333 + z2 * (0.2 + z2 * (0.14285714 + z2 * 0.11111111))))
        loss = m + p
        gid = t * _NSUB + s
        new_s = []
        new_c = []
        for g in range(_NGROUPS):
            mk = gid == g
            new_s.append(accs[g] + jnp.where(mk, loss, zeros))
            new_c.append(accc[g] + jnp.where(mk, ones, zeros))
        return tuple(new_s), tuple(new_c)

    def body(j, acc):
        accs, accc = acc
        for k in range(_UNROLL):
            accs, accc = one_chunk(j * _UNROLL + k, accs, accc)
        return accs, accc

    init = (tuple(zeros for _ in range(_NGROUPS)),
            tuple(zeros for _ in range(_NGROUPS)))
    accs, accc = lax.fori_loop(0, _CHUNKS // _UNROLL, body, init)

    for g in range(_NGROUPS):
        blk[pl.ds(g * 16, 16)] = accs[g]
        blk[pl.ds((_NGROUPS + g) * 16, 16)] = accc[g]
    pltpu.sync_copy(blk, shared.at[pl.ds(wid * 256, 256)])
    plsc.subcore_barrier()

    @pl.when(wid == 0)
    def _finalize():
        pltpu.sync_copy(shared, gath)
        resv = zeros
        for g in range(_NGROUPS):
            svec = gath[pl.ds(g * 16, 16)]
            cvec = gath[pl.ds((_NGROUPS + g) * 16, 16)]
            for t in range(1, _NT):
                svec = svec + gath[pl.ds(t * 256 + g * 16, 16)]
                cvec = cvec + gath[pl.ds(t * 256 + (_NGROUPS + g) * 16, 16)]
            cb = jnp.full((16,), jnp.sum(cvec), jnp.float32)
            resv = resv + jnp.where(cb > 0.0, svec / cb, zeros)
        outv[...] = jnp.full((16,), jnp.sum(resv), jnp.float32)
        pltpu.sync_copy(outv, out_hbm)


def kernel(logits, targets, subgroup_inf):
    flat = logits.reshape(128, 128, 2).transpose(0, 2, 1).reshape(-1)
    out = _sc_loss(flat, targets, subgroup_inf)
    return out[0]


# speedup vs baseline: 1.1264x; 1.1264x over previous
import functools

import jax
import jax.numpy as jnp
from jax import lax
from jax.experimental import pallas as pl
from jax.experimental.pallas import tpu as pltpu
from jax.experimental.pallas import tpu_sc as plsc

_BATCH = 16384
_NSUB = 4
_NGROUPS = 8
_NT = 16
_PER = _BATCH // _NT
_CHUNKS = _PER // 16
_UNROLL = 1

_mesh = plsc.VectorSubcoreMesh(
    core_axis_name="c", subcore_axis_name="s", num_cores=1)


@functools.partial(
    pl.kernel,
    out_type=jax.ShapeDtypeStruct((16,), jnp.float32),
    mesh=_mesh,
    scratch_types=[
        pltpu.VMEM((2 * _PER,), jnp.float32),
        pltpu.VMEM((_PER,), jnp.int32),
        pltpu.VMEM((_PER,), jnp.int32),
        pltpu.VMEM((256,), jnp.float32),
        pltpu.VMEM((_NT * 256,), jnp.float32),
        pltpu.VMEM((16,), jnp.float32),
        pltpu.VMEM_SHARED((_NT * 256,), jnp.float32),
        pltpu.SemaphoreType.DMA,
        pltpu.SemaphoreType.DMA,
        pltpu.SemaphoreType.DMA,
    ],
    compiler_params=pltpu.CompilerParams(
        needs_layout_passes=False, use_tc_tiling_on_sc=False),
)
def _sc_loss(logits_hbm, tgt_hbm, sg_hbm, out_hbm,
             lg_v, tv, sv, blk, gath, outv, shared, sem0, sem1, sem2):
    wid = lax.axis_index("s")
    base = wid * _PER

    c0 = pltpu.async_copy(logits_hbm.at[pl.ds(2 * base, 2 * _PER)], lg_v, sem0)
    c1 = pltpu.async_copy(tgt_hbm.at[pl.ds(base, _PER)], tv, sem1)
    c2 = pltpu.async_copy(sg_hbm.at[pl.ds(base, _PER)], sv, sem2)
    c0.wait()
    c1.wait()
    c2.wait()

    zeros = jnp.zeros((16,), jnp.float32)
    ones = jnp.ones((16,), jnp.float32)

    def one_chunk(i, accs, accc):
        t = tv[pl.ds(i * 16, 16)]
        s = sv[pl.ds(i * 16, 16)]
        off = (i // 8) * 256 + (i % 8) * 16
        la = lg_v[pl.ds(off, 16)]
        lb = lg_v[pl.ds(off + 128, 16)]
        d = (1 - 2 * t).astype(jnp.float32) * (la - lb)
        m = jnp.maximum(-d, 0.0)
        u = jnp.exp(-jnp.abs(d))
        z = u / (u + 2.0)
        z2 = z * z
        p = 2.0 * z * (1.0 + z2 * (0.33333333 + z2 * (0.2 + z2 * (0.14285714 + z2 * 0.11111111))))
        loss = m + p
        gid = t * _NSUB + s
        new_s = []
        new_c = []
        for g in range(_NGROUPS):
            mk = gid == g
            new_s.append(accs[g] + jnp.where(mk, loss, zeros))
            new_c.append(accc[g] + jnp.where(mk, ones, zeros))
        return tuple(new_s), tuple(new_c)

    def body(j, acc):
        accs, accc = acc
        for k in range(_UNROLL):
            accs, accc = one_chunk(j * _UNROLL + k, accs, accc)
        return accs, accc

    init = (tuple(zeros for _ in range(_NGROUPS)),
            tuple(zeros for _ in range(_NGROUPS)))
    accs, accc = lax.fori_loop(0, _CHUNKS // _UNROLL, body, init)

    for g in range(_NGROUPS):
        blk[pl.ds(g * 16, 16)] = accs[g]
        blk[pl.ds((_NGROUPS + g) * 16, 16)] = accc[g]
    pltpu.sync_copy(blk, shared.at[pl.ds(wid * 256, 256)])
    plsc.subcore_barrier()

    @pl.when(wid == 0)
    def _finalize():
        pltpu.sync_copy(shared, gath)
        def red_body(t, acc):
            tb = t * 256
            return tuple(v + gath[pl.ds(tb + r * 16, 16)]
                         for r, v in enumerate(acc))

        rows = lax.fori_loop(
            1, _NT, red_body,
            tuple(gath[pl.ds(r * 16, 16)] for r in range(2 * _NGROUPS)))
        resv = zeros
        for g in range(_NGROUPS):
            cb = jnp.full((16,), jnp.sum(rows[_NGROUPS + g]), jnp.float32)
            resv = resv + jnp.where(cb > 0.0, rows[g] / cb, zeros)
        outv[...] = jnp.full((16,), jnp.sum(resv), jnp.float32)
        pltpu.sync_copy(outv, out_hbm)


def kernel(logits, targets, subgroup_inf):
    flat = logits.reshape(128, 128, 2).transpose(0, 2, 1).reshape(-1)
    out = _sc_loss(flat, targets, subgroup_inf)
    return out[0]


# speedup vs baseline: 1.1314x; 1.0044x over previous
import functools

import jax
import jax.numpy as jnp
from jax import lax
from jax.experimental import pallas as pl
from jax.experimental.pallas import tpu as pltpu
from jax.experimental.pallas import tpu_sc as plsc

_BATCH = 16384
_NSUB = 4
_NGROUPS = 8
_NT = 16
_PER = _BATCH // _NT
_CHUNKS = _PER // 16
_UNROLL = 1

_mesh = plsc.VectorSubcoreMesh(
    core_axis_name="c", subcore_axis_name="s", num_cores=1)


@functools.partial(
    pl.kernel,
    out_type=jax.ShapeDtypeStruct((16,), jnp.float32),
    mesh=_mesh,
    scratch_types=[
        pltpu.VMEM((2 * _PER,), jnp.float32),
        pltpu.VMEM((_PER,), jnp.int32),
        pltpu.VMEM((_PER,), jnp.int32),
        pltpu.VMEM((256,), jnp.float32),
        pltpu.VMEM((_NT * 256,), jnp.float32),
        pltpu.VMEM((16,), jnp.float32),
        pltpu.VMEM_SHARED((_NT * 256,), jnp.float32),
        pltpu.SemaphoreType.DMA,
        pltpu.SemaphoreType.DMA,
        pltpu.SemaphoreType.DMA,
    ],
    compiler_params=pltpu.CompilerParams(
        needs_layout_passes=False, use_tc_tiling_on_sc=False),
)
def _sc_loss(logits_hbm, tgt_hbm, sg_hbm, out_hbm,
             lg_v, tv, sv, blk, gath, outv, shared, sem0, sem1, sem2):
    wid = lax.axis_index("s")
    base = wid * _PER

    c0 = pltpu.async_copy(logits_hbm.at[pl.ds(2 * base, 2 * _PER)], lg_v, sem0)
    c1 = pltpu.async_copy(tgt_hbm.at[pl.ds(base, _PER)], tv, sem1)
    c2 = pltpu.async_copy(sg_hbm.at[pl.ds(base, _PER)], sv, sem2)
    c0.wait()
    c1.wait()
    c2.wait()

    zeros = jnp.zeros((16,), jnp.float32)
    ones = jnp.ones((16,), jnp.float32)

    def one_chunk(i, accs, accc):
        t = tv[pl.ds(i * 16, 16)]
        s = sv[pl.ds(i * 16, 16)]
        off = (i // 8) * 256 + (i % 8) * 16
        la = lg_v[pl.ds(off, 16)]
        lb = lg_v[pl.ds(off + 128, 16)]
        d = (1 - 2 * t).astype(jnp.float32) * (la - lb)
        m = jnp.maximum(-d, 0.0)
        u = jnp.exp(-jnp.abs(d))
        z = u / (u + 2.0)
        z2 = z * z
        p = 2.0 * z * (1.0 + z2 * (0.33333333 + z2 * (0.2 + z2 * (0.14285714 + z2 * 0.11111111))))
        loss = m + p
        gid = t * _NSUB + s
        new_s = []
        new_c = []
        for g in range(_NGROUPS):
            mk = gid == g
            new_s.append(accs[g] + jnp.where(mk, loss, zeros))
            new_c.append(accc[g] + plsc.all_reduce_population_count(mk))
        return tuple(new_s), tuple(new_c)

    def body(j, acc):
        accs, accc = acc
        for k in range(_UNROLL):
            accs, accc = one_chunk(j * _UNROLL + k, accs, accc)
        return accs, accc

    izeros = jnp.zeros((16,), jnp.int32)
    init = (tuple(zeros for _ in range(_NGROUPS)),
            tuple(izeros for _ in range(_NGROUPS)))
    accs, accc = lax.fori_loop(0, _CHUNKS // _UNROLL, body, init)

    for g in range(_NGROUPS):
        blk[pl.ds(g * 16, 16)] = accs[g]
        blk[pl.ds((_NGROUPS + g) * 16, 16)] = accc[g].astype(jnp.float32)
    pltpu.sync_copy(blk, shared.at[pl.ds(wid * 256, 256)])
    plsc.subcore_barrier()

    @pl.when(wid == 0)
    def _finalize():
        pltpu.sync_copy(shared, gath)
        def red_body(t, acc):
            tb = t * 256
            return tuple(v + gath[pl.ds(tb + r * 16, 16)]
                         for r, v in enumerate(acc))

        rows = lax.fori_loop(
            1, _NT, red_body,
            tuple(gath[pl.ds(r * 16, 16)] for r in range(2 * _NGROUPS)))
        resv = zeros
        for g in range(_NGROUPS):
            cb = jnp.full((16,), jnp.sum(rows[_NGROUPS + g]) * 0.0625,
                          jnp.float32)
            resv = resv + jnp.where(cb > 0.0, rows[g] / cb, zeros)
        outv[...] = jnp.full((16,), jnp.sum(resv), jnp.float32)
        pltpu.sync_copy(outv, out_hbm)


def kernel(logits, targets, subgroup_inf):
    flat = logits.reshape(128, 128, 2).transpose(0, 2, 1).reshape(-1)
    out = _sc_loss(flat, targets, subgroup_inf)
    return out[0]
